# Initial kernel scaffold; baseline (speedup 1.0000x reference)
import jax
import jax.numpy as jnp
from jax.experimental import pallas as pl


def kernel(x, Wg, bg, W1, b1, W2, b2):
    raise NotImplementedError("write your pallas kernel here")


# speedup vs baseline: 1.4086x; 1.4086x over previous
import functools

import jax
import jax.numpy as jnp
from jax.experimental import pallas as pl
from jax.experimental.pallas import tpu as pltpu

NUM_EXPERTS = 8
TOP_K = 2
IN_DIM = 1024
HID_DIM = 2048
OUT_DIM = 1024

ROUTER_TM = 512
FFN_TM = 512


def _router_body(x_ref, wg_ref, bg_ref, weights_ref, idxpad_ref, usage_ref):
    t = pl.program_id(0)
    scores = jax.lax.dot_general(
        x_ref[...], wg_ref[...], (((1,), (1,)), ((), ())),
        preferred_element_type=jnp.float32)
    scores = scores + bg_ref[...]
    tm = scores.shape[0]
    iota_e = jax.lax.broadcasted_iota(jnp.int32, (tm, NUM_EXPERTS), 1)
    m0 = jnp.max(scores, axis=1, keepdims=True)
    idx0 = jnp.min(jnp.where(scores == m0, iota_e, NUM_EXPERTS), axis=1,
                   keepdims=True)
    masked = jnp.where(iota_e == idx0, -jnp.inf, scores)
    m1 = jnp.max(masked, axis=1, keepdims=True)
    idx1 = jnp.min(jnp.where(masked == m1, iota_e, NUM_EXPERTS), axis=1,
                   keepdims=True)
    e1 = jnp.exp(m1 - m0)
    denom = 1.0 + e1
    w0 = 1.0 / denom
    w1 = e1 / denom
    onehot0 = (iota_e == idx0)
    onehot1 = (iota_e == idx1)
    weights_ref[...] = (jnp.where(onehot0, w0, 0.0)
                        + jnp.where(onehot1, w1, 0.0))
    idxpad_ref[...] = jnp.where(iota_e == 0, idx0, idx1)
    usage = (jnp.sum(onehot0.astype(jnp.int32), axis=0, keepdims=True)
             + jnp.sum(onehot1.astype(jnp.int32), axis=0, keepdims=True))

    @pl.when(t == 0)
    def _():
        usage_ref[...] = jnp.zeros_like(usage_ref)

    usage_ref[...] += usage


def _ffn_body(x_ref, w1_ref, b1_ref, w2_ref, b2_ref, wts_ref, out_ref):
    e = pl.program_id(1)
    h = jax.lax.dot_general(
        x_ref[...], w1_ref[0], (((1,), (1,)), ((), ())),
        preferred_element_type=jnp.float32)
    h = jnp.maximum(h + b1_ref[0], 0.0)
    y = jax.lax.dot_general(
        h, w2_ref[0], (((1,), (1,)), ((), ())),
        preferred_element_type=jnp.float32)
    y = y + b2_ref[0]
    iota_e = jax.lax.broadcasted_iota(jnp.int32, wts_ref.shape, 1)
    w = jnp.sum(jnp.where(iota_e == e, wts_ref[...], 0.0), axis=1,
                keepdims=True)
    contrib = w * y

    @pl.when(e == 0)
    def _():
        out_ref[...] = jnp.zeros_like(out_ref)

    out_ref[...] += contrib


def kernel(x, Wg, bg, W1, b1, W2, b2):
    B, S, D = x.shape
    T = B * S
    xf = x.reshape(T, D)

    n_rt = T // ROUTER_TM
    weights, idxpad, usage2d = pl.pallas_call(
        _router_body,
        grid=(n_rt,),
        in_specs=[
            pl.BlockSpec((ROUTER_TM, D), lambda t: (t, 0)),
            pl.BlockSpec((NUM_EXPERTS, D), lambda t: (0, 0)),
            pl.BlockSpec((NUM_EXPERTS,), lambda t: (0,)),
        ],
        out_specs=[
            pl.BlockSpec((ROUTER_TM, NUM_EXPERTS), lambda t: (t, 0)),
            pl.BlockSpec((ROUTER_TM, NUM_EXPERTS), lambda t: (t, 0)),
            pl.BlockSpec((1, NUM_EXPERTS), lambda t: (0, 0)),
        ],
        out_shape=[
            jax.ShapeDtypeStruct((T, NUM_EXPERTS), jnp.float32),
            jax.ShapeDtypeStruct((T, NUM_EXPERTS), jnp.int32),
            jax.ShapeDtypeStruct((1, NUM_EXPERTS), jnp.int32),
        ],
    )(xf, Wg, bg)

    top_k_indices = idxpad[:, :TOP_K]
    expert_usage = usage2d.reshape(NUM_EXPERTS)

    n_ft = T // FFN_TM
    out_flat = pl.pallas_call(
        _ffn_body,
        grid=(n_ft, NUM_EXPERTS),
        in_specs=[
            pl.BlockSpec((FFN_TM, D), lambda t, e: (t, 0)),
            pl.BlockSpec((1, HID_DIM, D), lambda t, e: (e, 0, 0)),
            pl.BlockSpec((1, 1, HID_DIM), lambda t, e: (e, 0, 0)),
            pl.BlockSpec((1, OUT_DIM, HID_DIM), lambda t, e: (e, 0, 0)),
            pl.BlockSpec((1, 1, OUT_DIM), lambda t, e: (e, 0, 0)),
            pl.BlockSpec((FFN_TM, NUM_EXPERTS), lambda t, e: (t, 0)),
        ],
        out_specs=pl.BlockSpec((FFN_TM, OUT_DIM), lambda t, e: (t, 0)),
        out_shape=jax.ShapeDtypeStruct((T, OUT_DIM), jnp.float32),
    )(xf, W1, b1.reshape(NUM_EXPERTS, 1, HID_DIM),
      W2, b2.reshape(NUM_EXPERTS, 1, OUT_DIM), weights)

    output = out_flat.reshape(B, S, OUT_DIM)
    return output, weights, expert_usage, top_k_indices
